# Initial kernel scaffold; baseline (speedup 1.0000x reference)
import jax
import jax.numpy as jnp
from jax.experimental import pallas as pl


def kernel(x, Wg, Wgate, Wup, Wdown):
    raise NotImplementedError("write your pallas kernel here")


# speedup vs baseline: 1.3679x; 1.3679x over previous
import functools

import jax
import jax.numpy as jnp
from jax.experimental import pallas as pl
from jax.experimental.pallas import tpu as pltpu

T = 2048
H = 1024
I = 4096
E = 8
CAP = 768
BLOCK = 256
NB = (2 * T) // BLOCK + E
IT = 512
NI = I // IT


def _router_body(xf_ref, wg_ref, pos_ref, wts_ref, blk_ref, aux_ref):
    xf = xf_ref[...]
    wg = wg_ref[...]
    logits = jax.lax.dot_general(
        xf, wg, (((1,), (1,)), ((), ())),
        preferred_element_type=jnp.float32)
    m = jnp.max(logits, axis=-1, keepdims=True)
    p = jnp.exp(logits - m)
    probs = p / jnp.sum(p, axis=-1, keepdims=True)

    iota_e = jax.lax.broadcasted_iota(jnp.int32, (T, E), 1)
    m1 = jnp.max(probs, axis=-1, keepdims=True)
    i1 = jnp.min(jnp.where(probs == m1, iota_e, E), axis=-1, keepdims=True)
    oh1 = (iota_e == i1)
    pm = jnp.where(oh1, -1.0, probs)
    m2 = jnp.max(pm, axis=-1, keepdims=True)
    i2 = jnp.min(jnp.where(pm == m2, iota_e, E), axis=-1, keepdims=True)
    oh2 = (iota_e == i2)

    denom = m1 + m2 + 1e-8
    w0 = jnp.clip(m1 / denom, 1e-8, 10.0)
    w1 = jnp.clip(m2 / denom, 1e-8, 10.0)

    masks = jnp.concatenate(
        [oh1.astype(jnp.bfloat16), oh2.astype(jnp.bfloat16)], axis=1)
    r_i = jax.lax.broadcasted_iota(jnp.int32, (T, T), 0)
    c_i = jax.lax.broadcasted_iota(jnp.int32, (T, T), 1)
    tri = (c_i <= r_i).astype(jnp.bfloat16)
    csum = jax.lax.dot_general(
        tri, masks, (((1,), (0,)), ((), ())),
        preferred_element_type=jnp.float32)

    counts_raw = csum[T - 1:T, :]
    nkeep = jnp.minimum(counts_raw, float(CAP))
    dropped = jnp.sum(counts_raw - nkeep)
    counts_e = nkeep[:, :E] + nkeep[:, E:]
    total_e = counts_raw[:, :E] + counts_raw[:, E:]
    padded = jnp.floor((total_e + (BLOCK - 1)) / BLOCK) * BLOCK

    r8 = jax.lax.broadcasted_iota(jnp.int32, (E, E), 0)
    c8 = jax.lax.broadcasted_iota(jnp.int32, (E, E), 1)
    strict = (r8 < c8).astype(jnp.float32)
    off = jax.lax.dot_general(
        padded, strict, (((1,), (0,)), ((), ())),
        preferred_element_type=jnp.float32)
    end = off + padded

    oh1f = oh1.astype(jnp.float32)
    oh2f = oh2.astype(jnp.float32)
    rank0 = jnp.sum(csum[:, :E] * oh1f, axis=-1, keepdims=True)
    rank1 = jnp.sum(csum[:, E:] * oh2f, axis=-1, keepdims=True)
    off0 = jnp.sum(off * oh1f, axis=-1, keepdims=True)
    off1 = jnp.sum(off * oh2f, axis=-1, keepdims=True)
    cnt0_at2 = jnp.sum(counts_raw[:, :E] * oh2f, axis=-1, keepdims=True)
    pos0 = off0 + rank0 - 1.0
    pos1 = off1 + cnt0_at2 + rank1 - 1.0
    w0f = jnp.where(rank0 <= CAP, w0, 0.0)
    w1f = jnp.where(rank1 <= CAP, w1, 0.0)

    zpad_i = jnp.zeros((T, 126), jnp.int32)
    pos_ref[...] = jnp.concatenate(
        [pos0.astype(jnp.int32), pos1.astype(jnp.int32), zpad_i], axis=1)
    zpad_f = jnp.zeros((T, 126), jnp.float32)
    wts_ref[...] = jnp.concatenate([w0f, w1f, zpad_f], axis=1)

    jbase = (jax.lax.broadcasted_iota(jnp.int32, (1, 128), 1) * BLOCK
             ).astype(jnp.float32)
    acc = jnp.zeros((1, 128), jnp.int32)
    for e in range(E):
        end_e = jax.lax.slice(end, (0, e), (1, e + 1))
        acc = acc + jnp.where(end_e <= jbase, 1, 0)
    blk_ref[...] = jnp.minimum(acc, E - 1)

    importance = jnp.mean(probs, axis=0, keepdims=True)
    usage = counts_e / float(2 * T)
    aux = jnp.sum(usage * importance) * float(E)
    aux = jnp.where(dropped > 0, aux + dropped / float(T) * 0.1, aux)
    aux = jnp.minimum(aux, 1.0) * 0.001
    aux_ref[...] = jnp.full((1, 1), 1.0, jnp.float32) * aux


def _ffn_body(blk_ref, pos_ref, wts_ref, xf_ref, wgate_ref, wup_ref,
              wdown_ref, out_ref, x_sc, acc_sc):
    b = pl.program_id(0)
    i = pl.program_id(1)
    pos0 = pos_ref[:, 0:1]
    pos1 = pos_ref[:, 1:2]
    rvec = b * BLOCK + jax.lax.broadcasted_iota(jnp.int32, (1, BLOCK), 1)

    @pl.when(i == 0)
    def _gather():
        gt = (jnp.where(pos0 == rvec, 1.0, 0.0)
              + jnp.where(pos1 == rvec, 1.0, 0.0))
        x_sc[...] = jax.lax.dot_general(
            gt, xf_ref[...], (((0,), (0,)), ((), ())),
            preferred_element_type=jnp.float32)

    x = x_sc[...]
    g = jax.lax.dot_general(
        x, wgate_ref[0], (((1,), (1,)), ((), ())),
        preferred_element_type=jnp.float32)
    u = jax.lax.dot_general(
        x, wup_ref[0], (((1,), (1,)), ((), ())),
        preferred_element_type=jnp.float32)
    h = g * jax.nn.sigmoid(g) * u
    part = jax.lax.dot_general(
        h, wdown_ref[0], (((1,), (1,)), ((), ())),
        preferred_element_type=jnp.float32)

    @pl.when(i == 0)
    def _init_acc():
        acc_sc[...] = part

    @pl.when(i > 0)
    def _add_acc():
        acc_sc[...] = acc_sc[...] + part

    @pl.when(i == NI - 1)
    def _combine():
        w0 = wts_ref[:, 0:1]
        w1 = wts_ref[:, 1:2]
        s = (jnp.where(pos0 == rvec, w0, 0.0)
             + jnp.where(pos1 == rvec, w1, 0.0))
        contrib = jax.lax.dot_general(
            s, acc_sc[...], (((1,), (0,)), ((), ())),
            preferred_element_type=jnp.float32)

        @pl.when(b == 0)
        def _init_out():
            out_ref[...] = contrib

        @pl.when(b > 0)
        def _acc_out():
            out_ref[...] = out_ref[...] + contrib


@jax.jit
def kernel(x, Wg, Wgate, Wup, Wdown):
    B, S, Hd = x.shape
    xf = x.reshape(T, H)

    pos_a, wts_a, blk_a, aux_a = pl.pallas_call(
        _router_body,
        out_shape=[
            jax.ShapeDtypeStruct((T, 128), jnp.int32),
            jax.ShapeDtypeStruct((T, 128), jnp.float32),
            jax.ShapeDtypeStruct((1, 128), jnp.int32),
            jax.ShapeDtypeStruct((1, 1), jnp.float32),
        ],
    )(xf, Wg)

    blk = blk_a[0, :NB]

    grid_spec = pltpu.PrefetchScalarGridSpec(
        num_scalar_prefetch=1,
        grid=(NB, NI),
        in_specs=[
            pl.BlockSpec((T, 128), lambda b, i, s: (0, 0)),
            pl.BlockSpec((T, 128), lambda b, i, s: (0, 0)),
            pl.BlockSpec((T, H), lambda b, i, s: (0, 0)),
            pl.BlockSpec((1, IT, H), lambda b, i, s: (s[b], i, 0)),
            pl.BlockSpec((1, IT, H), lambda b, i, s: (s[b], i, 0)),
            pl.BlockSpec((1, H, IT), lambda b, i, s: (s[b], 0, i)),
        ],
        out_specs=pl.BlockSpec((T, H), lambda b, i, s: (0, 0)),
        scratch_shapes=[
            pltpu.VMEM((BLOCK, H), jnp.float32),
            pltpu.VMEM((BLOCK, H), jnp.float32),
        ],
    )
    out = pl.pallas_call(
        _ffn_body,
        grid_spec=grid_spec,
        out_shape=jax.ShapeDtypeStruct((T, H), jnp.float32),
    )(blk, pos_a, wts_a, xf, Wgate, Wup, Wdown)

    return out.reshape(B, S, Hd), aux_a[0, 0]
